# baseline (speedup 1.0000x reference)
import jax
import jax.numpy as jnp
from jax.experimental import pallas as pl
from jax.experimental.pallas import tpu as pltpu


def _body(x_ref, out_ref, len_ref):
    s = pl.program_id(0)
    x = x_ref[...]
    out_ref[...] = x
    rs = jnp.sum(x, axis=1)
    cnt = jnp.sum((rs != 0.0).astype(jnp.int32), axis=0)

    @pl.when(s == 0)
    def _init():
        len_ref[...] = jnp.zeros_like(len_ref)

    len_ref[...] += cnt[None, :]


def kernel(batch):
    S, B, D = batch.shape
    x2 = jnp.transpose(batch, (0, 2, 1))
    sS = 4
    out2, lengths2d = pl.pallas_call(
        _body,
        grid=(S // sS,),
        in_specs=[pl.BlockSpec((sS, D, B), lambda s: (s, 0, 0))],
        out_specs=[
            pl.BlockSpec((sS, D, B), lambda s: (s, 0, 0)),
            pl.BlockSpec((1, B), lambda s: (0, 0)),
        ],
        out_shape=[
            jax.ShapeDtypeStruct((S, D, B), jnp.float32),
            jax.ShapeDtypeStruct((1, B), jnp.int32),
        ],
        compiler_params=pltpu.CompilerParams(
            dimension_semantics=("arbitrary",),
        ),
    )(x2)
    states = jnp.transpose(out2, (2, 0, 1))
    return states, lengths2d.reshape(B)


# speedup vs baseline: 1.0178x; 1.0178x over previous
import jax
import jax.numpy as jnp
from jax.experimental import pallas as pl
from jax.experimental.pallas import tpu as pltpu


def _body(x_ref, out_ref, len_ref):
    s = pl.program_id(0)
    x = x_ref[...]
    out_ref[...] = x
    rs = jnp.sum(x, axis=1)
    cnt = jnp.sum((rs != 0.0).astype(jnp.int32), axis=0)

    @pl.when(s == 0)
    def _init():
        len_ref[...] = jnp.zeros_like(len_ref)

    len_ref[...] += cnt[None, :]


def kernel(batch):
    S, B, D = batch.shape
    x2 = jnp.transpose(batch, (0, 2, 1))
    sS = 10
    out2, lengths2d = pl.pallas_call(
        _body,
        grid=(S // sS,),
        in_specs=[pl.BlockSpec((sS, D, B), lambda s: (s, 0, 0))],
        out_specs=[
            pl.BlockSpec((sS, D, B), lambda s: (s, 0, 0)),
            pl.BlockSpec((1, B), lambda s: (0, 0)),
        ],
        out_shape=[
            jax.ShapeDtypeStruct((S, D, B), jnp.float32),
            jax.ShapeDtypeStruct((1, B), jnp.int32),
        ],
        compiler_params=pltpu.CompilerParams(
            dimension_semantics=("arbitrary",),
        ),
    )(x2)
    states = jnp.transpose(out2, (2, 0, 1))
    return states, lengths2d.reshape(B)
